# baseline (speedup 1.0000x reference)
import functools

import jax
import jax.numpy as jnp
from jax import lax
from jax.experimental import pallas as pl
from jax.experimental.pallas import tpu as pltpu
from jax.experimental.pallas import tpu_sc as plsc

EMB = 64
SCALE = 8.0

NUM_CORES = 2
NUM_SUBCORES = 16
NUM_WORKERS = NUM_CORES * NUM_SUBCORES

IDX_W = 128
CHUNK = 1024
ROWS_PER_CHUNK = CHUNK // IDX_W


def _body(table_hbm, tok_hbm, out_hbm, idx_v, rows_v, sem):
    wid = lax.axis_index("s") * NUM_CORES + lax.axis_index("c")
    n_tok = out_hbm.shape[0]
    tok_per_w = n_tok // NUM_WORKERS
    chunks = tok_per_w // CHUNK
    base_tok = wid * tok_per_w
    base_row = base_tok // IDX_W

    def chunk_body(g, carry):
        row0 = pl.multiple_of(base_row + g * ROWS_PER_CHUNK, 8)
        tok0 = pl.multiple_of(base_tok + g * CHUNK, 8)
        pltpu.sync_copy(tok_hbm.at[pl.ds(row0, ROWS_PER_CHUNK)], idx_v)
        cps = [
            pltpu.async_copy(
                table_hbm.at[idx_v.at[j]],
                rows_v.at[pl.ds(j * IDX_W, IDX_W)],
                sem,
            )
            for j in range(ROWS_PER_CHUNK)
        ]
        for cp in cps:
            cp.wait()

        def scale_body(r, c2):
            for rr in range(4):
                for cc in range(EMB // 16):
                    sl = (r * 4 + rr, pl.ds(cc * 16, 16))
                    rows_v[sl] = rows_v[sl] * SCALE
            return c2

        lax.fori_loop(0, CHUNK // 4, scale_body, 0, unroll=False)

        pltpu.sync_copy(rows_v, out_hbm.at[pl.ds(tok0, CHUNK)])
        return carry

    lax.fori_loop(0, chunks, chunk_body, 0, unroll=False)


@functools.partial(jax.jit, static_argnames=())
def kernel(tokens, table):
    b, s = tokens.shape
    n = b * s
    tok2d = tokens.reshape(n // IDX_W, IDX_W).astype(jnp.int32)

    mesh = plsc.VectorSubcoreMesh(
        core_axis_name="c", subcore_axis_name="s",
        num_cores=NUM_CORES, num_subcores=NUM_SUBCORES,
    )
    out = pl.kernel(
        _body,
        out_type=jax.ShapeDtypeStruct((n, EMB), jnp.float32),
        mesh=mesh,
        compiler_params=pltpu.CompilerParams(use_tc_tiling_on_sc=False),
        scratch_types=[
            pltpu.VMEM((ROWS_PER_CHUNK, IDX_W), jnp.int32),
            pltpu.VMEM((CHUNK, EMB), jnp.float32),
            pltpu.SemaphoreType.DMA,
        ],
    )(table, tok2d)
    return out.reshape(b, s, EMB)


# speedup vs baseline: 1.0618x; 1.0618x over previous
import jax
import jax.numpy as jnp
from jax import lax
from jax.experimental import pallas as pl
from jax.experimental.pallas import tpu as pltpu
from jax.experimental.pallas import tpu_sc as plsc

EMB = 64
SCALE = 8.0

NUM_CORES = 2
NUM_SUBCORES = 16
NUM_WORKERS = NUM_CORES * NUM_SUBCORES

IDX_W = 128
STEP = 512
GATHERS = STEP // IDX_W
NBUF = 3
N_TOK = 4096 * 200
TOK_PER_W = N_TOK // NUM_WORKERS
STEPS = TOK_PER_W // STEP
IDX_ROWS = TOK_PER_W // IDX_W


def _body(table_hbm, tok_hbm, out_hbm,
          idx_all, rows0, rows1, rows2,
          gsem0, gsem1, gsem2, osem0, osem1, osem2):
    rows = (rows0, rows1, rows2)
    gsem = (gsem0, gsem1, gsem2)
    osem = (osem0, osem1, osem2)

    wid = lax.axis_index("s") * NUM_CORES + lax.axis_index("c")
    base_tok = wid * TOK_PER_W
    base_row = base_tok // IDX_W

    pltpu.sync_copy(tok_hbm.at[pl.ds(pl.multiple_of(base_row, 8), IDX_ROWS)],
                    idx_all)

    def fire_gathers(h, d):
        for j in range(GATHERS):
            pltpu.async_copy(
                table_hbm.at[idx_all.at[h * GATHERS + j]],
                rows[d].at[pl.ds(j * IDX_W, IDX_W)],
                gsem[d],
            )

    def wait_gathers(d):
        pltpu.make_async_copy(
            table_hbm.at[pl.ds(0, STEP)], rows[d], gsem[d]).wait()

    def fire_writeout(h, d):
        tok0 = pl.multiple_of(base_tok + h * STEP, 8)
        pltpu.async_copy(rows[d], out_hbm.at[pl.ds(tok0, STEP)], osem[d])

    def wait_writeout(d):
        pltpu.make_async_copy(
            rows[d], out_hbm.at[pl.ds(0, STEP)], osem[d]).wait()

    def scale(d):
        def sbody(r, c):
            for rr in range(8):
                for cc in range(EMB // 16):
                    sl = (r * 8 + rr, pl.ds(cc * 16, 16))
                    rows[d][sl] = rows[d][sl] * SCALE
            return c

        lax.fori_loop(0, STEP // 8, sbody, 0, unroll=False)

    def pipe_step(h, d, first_round):
        wait_gathers(d)
        d2 = (d + 1) % NBUF
        if not first_round:
            wait_writeout(d2)
        fire_gathers(h + 1, d2)
        scale(d)
        fire_writeout(h, d)

    fire_gathers(0, 0)
    pipe_step(0, 0, True)
    pipe_step(1, 1, True)
    pipe_step(2, 2, False)

    def loop_body(i, c):
        h = i * NBUF
        pipe_step(h, 0, False)
        pipe_step(h + 1, 1, False)
        pipe_step(h + 2, 2, False)
        return c

    lax.fori_loop(1, STEPS // NBUF, loop_body, 0, unroll=False)

    h = NBUF * (STEPS // NBUF)
    wait_gathers(0)
    wait_writeout(1)
    fire_gathers(h + 1, 1)
    scale(0)
    fire_writeout(h, 0)

    wait_gathers(1)
    scale(1)
    fire_writeout(h + 1, 1)

    wait_writeout(2)
    wait_writeout(0)
    wait_writeout(1)


def kernel(tokens, table):
    b, s = tokens.shape
    n = b * s
    tok2d = tokens.reshape(n // IDX_W, IDX_W).astype(jnp.int32)

    mesh = plsc.VectorSubcoreMesh(
        core_axis_name="c", subcore_axis_name="s",
        num_cores=NUM_CORES, num_subcores=NUM_SUBCORES,
    )
    out = pl.kernel(
        _body,
        out_type=jax.ShapeDtypeStruct((n, EMB), jnp.float32),
        mesh=mesh,
        compiler_params=pltpu.CompilerParams(use_tc_tiling_on_sc=False),
        scratch_types=[
            pltpu.VMEM((IDX_ROWS, IDX_W), jnp.int32),
            pltpu.VMEM((STEP, EMB), jnp.float32),
            pltpu.VMEM((STEP, EMB), jnp.float32),
            pltpu.VMEM((STEP, EMB), jnp.float32),
            pltpu.SemaphoreType.DMA,
            pltpu.SemaphoreType.DMA,
            pltpu.SemaphoreType.DMA,
            pltpu.SemaphoreType.DMA,
            pltpu.SemaphoreType.DMA,
            pltpu.SemaphoreType.DMA,
        ],
    )(table, tok2d)
    return out.reshape(b, s, EMB)
